# baseline (speedup 1.0000x reference)
import jax
import jax.numpy as jnp
from jax import lax
from jax.experimental import pallas as pl
from jax.experimental.pallas import tpu as pltpu
from jax.experimental.pallas import tpu_sc as plsc

_N = 50000
_E = 800000
_IRR = 32
_DPER = 3
_AVG = 16.0
_H = 64

_CORES = 2
_TILES = 16
_SUB = 128
_NSUB = 8
_CHUNK = _SUB * _NSUB
_MACROS = 49
_EPT = _CHUNK * _MACROS
_EPAD = _EPT * _TILES
_ROWS = _EPAD // _SUB
_NPAD = 50176
_NPT = _NPAD // _TILES
_NQ = _NPT // _SUB
_NTAIL = _NPT - _NQ * _SUB
_MLP_B = 4096


def _mlp_body(r_ref, w1_ref, b1_ref, w2_ref, b2_ref, w3_ref, b3_ref, out_ref):
    i = pl.program_id(0)
    r = r_ref[...]
    h = jax.nn.silu(r * w1_ref[...] + b1_ref[...])
    h = jax.nn.silu(
        jnp.dot(h, w2_ref[...], preferred_element_type=jnp.float32) + b2_ref[...]
    )
    mix = jnp.dot(h, w3_ref[...], preferred_element_type=jnp.float32) + b3_ref[...]
    mix = mix * (1.0 / _AVG)
    eidx = i * _MLP_B + lax.broadcasted_iota(jnp.int32, (_MLP_B, 1), 0)
    mix = jnp.where(eidx < _E, mix, 0.0)
    out_ref[0, :, :] = mix[:, :16]
    out_ref[1, :, :] = mix[:, 16:]


def _take16(v, idx):
    dnums = lax.GatherDimensionNumbers(
        offset_dims=(), collapsed_slice_dims=(0,), start_index_map=(0,))
    return lax.gather(v, idx[:, None], dnums, (1,),
                      mode=lax.GatherScatterMode.PROMISE_IN_BOUNDS)


def _sc_body(snd_hbm, rcv_hbm, mix_hbm, nf_hbm, out_hbm,
             snd_v, rcv_v, idx0_v, idx1_v, mix_v, nf0_v, nf1_v,
             zero_v, stage_v, ibuf_v, idxw_v, idxw_t, acc0_sh,
             sem_g, sem_s):
    c = lax.axis_index("c")
    s = lax.axis_index("s")
    nbase = s * _NPT
    iota = lax.iota(jnp.int32, 16)
    c3 = jnp.zeros((16,), jnp.int32) + 3 * c

    def zfill(q, carry):
        zero_v[q, :] = jnp.zeros((16,), jnp.float32)
        return carry
    lax.fori_loop(0, _SUB, zfill, 0, unroll=8)
    for g in range(8):
        ibuf_v[pl.ds(g * 16, 16)] = (iota + 16 * g) * 6

    perms = [lax.div(iota + 16 * k, 3) for k in range(3)]

    def zero_acc(acc):
        for q in range(_NQ):
            pltpu.sync_copy(zero_v, acc.at[pl.ds(nbase + q * _SUB, _SUB)])
        pltpu.sync_copy(zero_v.at[pl.ds(0, _NTAIL)],
                        acc.at[pl.ds(nbase + _NQ * _SUB, _NTAIL)])

    def writeout(acc, k):
        for q in range(_NQ + 1):
            rows = _SUB if q < _NQ else _NTAIL
            idx_ref = idxw_v if q < _NQ else idxw_t
            r0 = nbase + q * _SUB
            base = jnp.zeros((16,), jnp.int32) + (6 * r0 + 3 * c + k)
            for g in range(rows // 16):
                idx_ref[pl.ds(g * 16, 16)] = ibuf_v[pl.ds(g * 16, 16)] + base
            pltpu.sync_copy(acc.at[pl.ds(r0, rows)],
                            stage_v.at[pl.ds(0, rows)])
            pltpu.sync_copy(stage_v.at[pl.ds(0, rows)], out_hbm.at[idx_ref])

    def run_pass(blocks, accs):
        def macro(m, carry):
            row0 = s * (_EPT // _SUB) + m * _NSUB
            pltpu.sync_copy(snd_hbm.at[pl.ds(row0, _NSUB)], snd_v)
            pltpu.sync_copy(rcv_hbm.at[pl.ds(row0, _NSUB)], rcv_v)
            pltpu.sync_copy(mix_hbm.at[pl.ds(c * _EPAD + row0 * _SUB, _CHUNK)],
                            mix_v)

            idx_bufs = [idx0_v, idx1_v][:len(blocks)]
            def mkidx(j, cr):
                for g in range(_SUB // 16):
                    sl = pl.ds(g * 16, 16)
                    base = snd_v[j, sl] * 6 + c3
                    for k, ib in zip(blocks, idx_bufs):
                        ib[j, sl] = base + k
                return cr
            lax.fori_loop(0, _NSUB, mkidx, 0)

            nf_bufs = [nf0_v, nf1_v][:len(blocks)]
            cps = [
                pltpu.async_copy(nf_hbm.at[ib.at[j]],
                                 nb.at[pl.ds(j * _SUB, _SUB)], sem_g)
                for ib, nb in zip(idx_bufs, nf_bufs)
                for j in range(_NSUB)
            ]
            for cp in cps:
                cp.wait()

            def mul(e, cr):
                mrow = mix_v[e, :]
                for k, nb in zip(blocks, nf_bufs):
                    nb[e, :] = nb[e, :] * _take16(mrow, perms[k])
                return cr
            lax.fori_loop(0, _CHUNK, mul, 0, unroll=8)

            cps2 = [
                pltpu.async_copy(nb.at[pl.ds(j * _SUB, _SUB)],
                                 acc.at[rcv_v.at[j]], sem_s, add=True)
                for nb, acc in zip(nf_bufs, accs)
                for j in range(_NSUB)
            ]
            for cp in cps2:
                cp.wait()
            return carry
        lax.fori_loop(0, _MACROS, macro, 0)

    for k in range(3):
        zero_acc(acc0_sh)
        plsc.subcore_barrier()
        run_pass([k], [acc0_sh])
        plsc.subcore_barrier()
        writeout(acc0_sh, k)


def kernel(vectors, node_feats, radial_embedding, senders, receivers,
           W1, b1, W2, b2, W3, b3):
    pad = _EPAD - _E
    r_pad = jnp.concatenate(
        [radial_embedding, jnp.zeros((pad, 1), jnp.float32)], axis=0)
    mix6 = pl.pallas_call(
        _mlp_body,
        grid=(_EPAD // _MLP_B,),
        in_specs=[
            pl.BlockSpec((_MLP_B, 1), lambda i: (i, 0)),
            pl.BlockSpec((1, _H), lambda i: (0, 0)),
            pl.BlockSpec((1, _H), lambda i: (0, 0)),
            pl.BlockSpec((_H, _H), lambda i: (0, 0)),
            pl.BlockSpec((1, _H), lambda i: (0, 0)),
            pl.BlockSpec((_H, _IRR), lambda i: (0, 0)),
            pl.BlockSpec((1, _IRR), lambda i: (0, 0)),
        ],
        out_specs=pl.BlockSpec((_CORES, _MLP_B, 16), lambda i: (0, i, 0)),
        out_shape=jax.ShapeDtypeStruct((_CORES, _EPAD, 16), jnp.float32),
    )(r_pad, W1, b1.reshape(1, _H), W2, b2.reshape(1, _H),
      W3, b3.reshape(1, _IRR))
    mix_flat = mix6.reshape(_CORES * _EPAD, 16)

    nf_flat = node_feats.reshape(6 * _N, 16)
    zpad = jnp.zeros((pad,), jnp.int32)
    snd = jnp.concatenate([senders, zpad]).reshape(_ROWS, _SUB)
    rcv = jnp.concatenate([receivers, zpad]).reshape(_ROWS, _SUB)

    mesh = plsc.VectorSubcoreMesh(core_axis_name="c", subcore_axis_name="s")
    out6 = pl.kernel(
        _sc_body,
        out_type=jax.ShapeDtypeStruct((6 * _NPAD, 16), jnp.float32),
        mesh=mesh,
        compiler_params=pltpu.CompilerParams(use_tc_tiling_on_sc=False),
        scratch_types=[
            pltpu.VMEM((_NSUB, _SUB), jnp.int32),
            pltpu.VMEM((_NSUB, _SUB), jnp.int32),
            pltpu.VMEM((_NSUB, _SUB), jnp.int32),
            pltpu.VMEM((_NSUB, _SUB), jnp.int32),
            pltpu.VMEM((_CHUNK, 16), jnp.float32),
            pltpu.VMEM((_CHUNK, 16), jnp.float32),
            pltpu.VMEM((_CHUNK, 16), jnp.float32),
            pltpu.VMEM((_SUB, 16), jnp.float32),
            pltpu.VMEM((_SUB, 16), jnp.float32),
            pltpu.VMEM((_SUB,), jnp.int32),
            pltpu.VMEM((_SUB,), jnp.int32),
            pltpu.VMEM((_NTAIL,), jnp.int32),
            pltpu.VMEM_SHARED((_NPAD, 16), jnp.float32),
            pltpu.SemaphoreType.DMA,
            pltpu.SemaphoreType.DMA,
        ],
    )(snd, rcv, mix_flat, nf_flat)

    out = out6[:6 * _N].reshape(_N, _IRR, _DPER)
    return out


# speedup vs baseline: 1.0369x; 1.0369x over previous
import jax
import jax.numpy as jnp
from jax import lax
from jax.experimental import pallas as pl
from jax.experimental.pallas import tpu as pltpu
from jax.experimental.pallas import tpu_sc as plsc

_N = 50000
_E = 800000
_IRR = 32
_DPER = 3
_AVG = 16.0
_H = 64

_CORES = 2
_TILES = 16
_SUB = 128
_NSUB = 8
_CHUNK = _SUB * _NSUB
_ROWS = _E // _SUB
_RPT = _ROWS // _TILES
_REM = _ROWS - _RPT * _TILES
_MACROS = _RPT // _NSUB
_TAIL0 = _RPT - _MACROS * _NSUB
_NPT_A = 3128
_NPT_B = _N - 15 * _NPT_A
_WQ = 24
_MLP_B = 4096
_MLP_G = 196
_EPAD = _MLP_G * _MLP_B


def _mlp_body(r_ref, w1_ref, b1_ref, w2_ref, b2_ref, w3_ref, b3_ref, out_ref):
    i = pl.program_id(0)
    r = r_ref[...]
    h = jax.nn.silu(r * w1_ref[...] + b1_ref[...])
    h = jax.nn.silu(
        jnp.dot(h, w2_ref[...], preferred_element_type=jnp.float32) + b2_ref[...]
    )
    mix = jnp.dot(h, w3_ref[...], preferred_element_type=jnp.float32) + b3_ref[...]
    mix = mix * (1.0 / _AVG)
    eidx = i * _MLP_B + lax.broadcasted_iota(jnp.int32, (_MLP_B, 1), 0)
    mix = jnp.where(eidx < _E, mix, 0.0)
    out_ref[0, :, :] = mix[:, :16]
    out_ref[1, :, :] = mix[:, 16:]


def _take16(v, idx):
    dnums = lax.GatherDimensionNumbers(
        offset_dims=(), collapsed_slice_dims=(0,), start_index_map=(0,))
    return lax.gather(v, idx[:, None], dnums, (1,),
                      mode=lax.GatherScatterMode.PROMISE_IN_BOUNDS)


def _sc_body(snd_hbm, rcv_hbm, mix_hbm, nf_hbm, out_hbm,
             snd_v, rcv_v, idx_v, mix_v, nf_v,
             zero_v, stage_v, ibuf_v, idxw_v, acc_sh, sem_g, sem_s):
    c = lax.axis_index("c")
    s = lax.axis_index("s")
    iota = lax.iota(jnp.int32, 16)
    c3 = jnp.zeros((16,), jnp.int32) + 3 * c

    row_base = s * _RPT + jnp.minimum(s, _REM)
    n_tail = _TAIL0 + jnp.where(s < _REM, 1, 0)
    nbase = s * _NPT_A
    ncnt = jnp.where(s < _TILES - 1, _NPT_A, _NPT_B)

    def zfill(q, carry):
        zero_v[q, :] = jnp.zeros((16,), jnp.float32)
        return carry
    lax.fori_loop(0, _SUB, zfill, 0, unroll=8)
    for g in range(8):
        ibuf_v[pl.ds(g * 16, 16)] = (iota + 16 * g) * 6

    perms = [lax.div(iota + 16 * k, 3) for k in range(3)]

    def zero_acc():
        for q in range(_WQ):
            pltpu.sync_copy(zero_v, acc_sh.at[pl.ds(nbase + q * _SUB, _SUB)])
        pltpu.sync_copy(zero_v, acc_sh.at[pl.ds(nbase + ncnt - _SUB, _SUB)])

    def writeout(k):
        def one(off):
            base = jnp.zeros((16,), jnp.int32) + (6 * (nbase + off) + 3 * c + k)
            for g in range(8):
                sl = pl.ds(g * 16, 16)
                idxw_v[sl] = ibuf_v[sl] + base
            pltpu.sync_copy(acc_sh.at[pl.ds(nbase + off, _SUB)], stage_v)
            pltpu.sync_copy(stage_v, out_hbm.at[idxw_v])
        for q in range(_WQ):
            one(q * _SUB)
        one(ncnt - _SUB)

    def do_rows(row0, nrows_static, k):
        nsub = nrows_static
        pltpu.sync_copy(snd_hbm.at[pl.ds(row0, nsub)],
                        snd_v.at[pl.ds(0, nsub)])
        pltpu.sync_copy(rcv_hbm.at[pl.ds(row0, nsub)],
                        rcv_v.at[pl.ds(0, nsub)])
        pltpu.sync_copy(mix_hbm.at[pl.ds(c * _EPAD + row0 * _SUB, nsub * _SUB)],
                        mix_v.at[pl.ds(0, nsub * _SUB)])

        def mkidx(j, cr):
            for g in range(_SUB // 16):
                sl = pl.ds(g * 16, 16)
                idx_v[j, sl] = snd_v[j, sl] * 6 + c3 + k
            return cr
        lax.fori_loop(0, nsub, mkidx, 0)

        cps = [
            pltpu.async_copy(nf_hbm.at[idx_v.at[j]],
                             nf_v.at[pl.ds(j * _SUB, _SUB)], sem_g)
            for j in range(nsub)
        ]
        for cp in cps:
            cp.wait()

        def mul(e, cr):
            nf_v[e, :] = nf_v[e, :] * _take16(mix_v[e, :], perms[k])
            return cr
        lax.fori_loop(0, nsub * _SUB, mul, 0, unroll=8)

        cps2 = [
            pltpu.async_copy(nf_v.at[pl.ds(j * _SUB, _SUB)],
                             acc_sh.at[rcv_v.at[j]], sem_s, add=True)
            for j in range(nsub)
        ]
        for cp in cps2:
            cp.wait()

    for k in range(3):
        zero_acc()
        plsc.subcore_barrier()

        def macro(m, carry):
            do_rows(row_base + m * _NSUB, _NSUB, k)
            return carry
        lax.fori_loop(0, _MACROS, macro, 0)

        def tail(t, carry):
            do_rows(row_base + _MACROS * _NSUB + t, 1, k)
            return carry
        lax.fori_loop(0, n_tail, tail, 0)

        plsc.subcore_barrier()
        writeout(k)
        plsc.subcore_barrier()


def kernel(vectors, node_feats, radial_embedding, senders, receivers,
           W1, b1, W2, b2, W3, b3):
    mix6 = pl.pallas_call(
        _mlp_body,
        grid=(_MLP_G,),
        in_specs=[
            pl.BlockSpec((_MLP_B, 1), lambda i: (i, 0)),
            pl.BlockSpec((1, _H), lambda i: (0, 0)),
            pl.BlockSpec((1, _H), lambda i: (0, 0)),
            pl.BlockSpec((_H, _H), lambda i: (0, 0)),
            pl.BlockSpec((1, _H), lambda i: (0, 0)),
            pl.BlockSpec((_H, _IRR), lambda i: (0, 0)),
            pl.BlockSpec((1, _IRR), lambda i: (0, 0)),
        ],
        out_specs=pl.BlockSpec((_CORES, _MLP_B, 16), lambda i: (0, i, 0)),
        out_shape=jax.ShapeDtypeStruct((_CORES, _EPAD, 16), jnp.float32),
    )(radial_embedding, W1, b1.reshape(1, _H), W2, b2.reshape(1, _H),
      W3, b3.reshape(1, _IRR))
    mix_flat = mix6.reshape(_CORES * _EPAD, 16)

    nf_flat = node_feats.reshape(6 * _N, 16)
    snd = senders.reshape(_ROWS, _SUB)
    rcv = receivers.reshape(_ROWS, _SUB)

    mesh = plsc.VectorSubcoreMesh(core_axis_name="c", subcore_axis_name="s")
    out6 = pl.kernel(
        _sc_body,
        out_type=jax.ShapeDtypeStruct((6 * _N, 16), jnp.float32),
        mesh=mesh,
        compiler_params=pltpu.CompilerParams(use_tc_tiling_on_sc=False),
        scratch_types=[
            pltpu.VMEM((_NSUB, _SUB), jnp.int32),
            pltpu.VMEM((_NSUB, _SUB), jnp.int32),
            pltpu.VMEM((_NSUB, _SUB), jnp.int32),
            pltpu.VMEM((_CHUNK, 16), jnp.float32),
            pltpu.VMEM((_CHUNK, 16), jnp.float32),
            pltpu.VMEM((_SUB, 16), jnp.float32),
            pltpu.VMEM((_SUB, 16), jnp.float32),
            pltpu.VMEM((_SUB,), jnp.int32),
            pltpu.VMEM((_SUB,), jnp.int32),
            pltpu.VMEM_SHARED((_N, 16), jnp.float32),
            pltpu.SemaphoreType.DMA,
            pltpu.SemaphoreType.DMA,
        ],
    )(snd, rcv, mix_flat, nf_flat)

    return out6.reshape(_N, _IRR, _DPER)
